# baseline (speedup 1.0000x reference)
import functools

import jax
import jax.numpy as jnp
from jax import lax
from jax.experimental import pallas as pl
from jax.experimental.pallas import tpu as pltpu
from jax.experimental.pallas import tpu_sc as plsc

N = 10000
D = 128
H = 16
C = 7
E = 320000
K = 4
E_EGO = 40000

NC = 2
NS = 16
CHUNK = 128
NW = NC * NS

ACC = 10240
RPS = ACC // NS

E_STREAMS = 128
E_KK = 2500
EGO_STREAMS = 128
EGO_KK = 1250

ZUNROLL = 16


def _make_agg(nstreams, kk):
    base = nstreams // NW
    rem = nstreams - base * NW
    cap = base + (1 if rem else 0)

    @functools.partial(
        pl.kernel,
        out_type=jax.ShapeDtypeStruct((NC, ACC, H), jnp.float32),
        mesh=plsc.VectorSubcoreMesh(core_axis_name="c", subcore_axis_name="s",
                                    num_cores=NC, num_subcores=NS),
        scratch_types=[
            pltpu.VMEM((cap, kk), jnp.int32),
            pltpu.VMEM((cap, kk), jnp.int32),
            pltpu.VMEM((kk, H), jnp.float32),
            pltpu.VMEM((kk, H), jnp.float32),
            pltpu.VMEM((RPS, H), jnp.float32),
            pltpu.VMEM_SHARED((ACC, H), jnp.float32),
            pltpu.SemaphoreType.DMA,
            pltpu.SemaphoreType.DMA,
            pltpu.SemaphoreType.DMA,
            pltpu.SemaphoreType.DMA,
        ],
        compiler_params=pltpu.CompilerParams(use_tc_tiling_on_sc=False),
    )
    def agg(rows_hbm, src_hbm, dst_hbm, out_hbm,
            src_v, dst_v, buf0, buf1, zbuf, acc, sem0, sem1, semi0, semi1):
        c = lax.axis_index("c")
        s = lax.axis_index("s")
        wid = c * NS + s
        n_my = jnp.where(wid < rem, base + 1, base)

        pltpu.async_copy(src_hbm.at[pl.ds(wid * base, base)],
                         src_v.at[pl.ds(0, base)], semi0)
        pltpu.async_copy(dst_hbm.at[pl.ds(wid * base, base)],
                         dst_v.at[pl.ds(0, base)], semi1)

        for i in range(ZUNROLL):
            zbuf[i] = jnp.zeros((H,), jnp.float32)

        def zero_body(i, _):
            for r in range(ZUNROLL):
                zbuf[i * ZUNROLL + r] = jnp.zeros((H,), jnp.float32)
            return 0

        lax.fori_loop(1, RPS // ZUNROLL, zero_body, 0)
        pltpu.sync_copy(zbuf, acc.at[pl.ds(s * RPS, RPS)])

        pltpu.make_async_copy(src_hbm.at[pl.ds(wid * base, base)],
                              src_v.at[pl.ds(0, base)], semi0).wait()
        pltpu.make_async_copy(dst_hbm.at[pl.ds(wid * base, base)],
                              dst_v.at[pl.ds(0, base)], semi1).wait()
        if rem:
            @pl.when(wid < rem)
            def _():
                pltpu.sync_copy(src_hbm.at[pl.ds(NW * base + wid, 1)],
                                src_v.at[pl.ds(base, 1)])
                pltpu.sync_copy(dst_hbm.at[pl.ds(NW * base + wid, 1)],
                                dst_v.at[pl.ds(base, 1)])

        plsc.subcore_barrier()

        def gather(j, buf, sem):
            pltpu.async_copy(rows_hbm.at[src_v.at[j]], buf, sem)

        def gwait(j, buf, sem):
            pltpu.make_async_copy(rows_hbm.at[src_v.at[j]], buf, sem).wait()

        def scatter(j, buf):
            pltpu.sync_copy(buf, acc.at[dst_v.at[j]], add=True)

        gather(0, buf0, sem0)

        def pair_body(p, _):
            j = 2 * p
            gather(j + 1, buf1, sem1)
            gwait(j, buf0, sem0)
            scatter(j, buf0)

            @pl.when(j + 2 < n_my)
            def _():
                gather(j + 2, buf0, sem0)

            gwait(j + 1, buf1, sem1)
            scatter(j + 1, buf1)
            return 0

        lax.fori_loop(0, n_my // 2, pair_body, 0)

        @pl.when(n_my % 2 == 1)
        def _():
            gwait(n_my - 1, buf0, sem0)
            scatter(n_my - 1, buf0)

        plsc.subcore_barrier()
        pltpu.sync_copy(acc.at[pl.ds(s * RPS, RPS)],
                        out_hbm.at[c, pl.ds(s * RPS, RPS)])

    return agg


@functools.cache
def _get_agg(nstreams, kk):
    return _make_agg(nstreams, kk)


NP = N // 8
GP = ACC // 8


def _lane_group_iota():
    return lax.broadcasted_iota(jnp.int32, (NP, 128), 1) % H


def _roll_left(x, k):
    return jnp.concatenate([x[:, k:], x[:, :k]], axis=1)


def _roll_right(x, k):
    return jnp.concatenate([x[:, -k:], x[:, :-k]], axis=1)


def _group_broadcast_lane0(s, lane):
    for k in (1, 2, 4, 8):
        s = jnp.where(lane >= k, _roll_right(s, k), s)
    return s


def _tc_pre_body(x_ref, kw1e_ref, kw1a_ref, b1et_ref, upre_ref, z_ref):
    kw1a = kw1a_ref[...]
    kp = jnp.dot(kw1e_ref[...], kw1a, preferred_element_type=jnp.float32)
    xv = x_ref[...]
    z = jnp.dot(xv, kp, preferred_element_type=jnp.float32)
    a = jnp.dot(xv, kw1a, preferred_element_type=jnp.float32)
    c1t = jnp.dot(b1et_ref[...], kw1a, preferred_element_type=jnp.float32)
    z_ref[...] = z
    upre_ref[...] = a + z + c1t


def _tc_combine_body(base_ref, g_ref, out_ref):
    out_ref[...] = base_ref[...] + 0.25 * (g_ref[0, :NP, :] + g_ref[1, :NP, :])


def _tc_mid_body(u_ref, e_ref, b1it_ref, kw2e_ref, b2et_ref, w_ref, s3_ref):
    h2 = u_ref[...] + e_ref[0, :NP, :] + e_ref[1, :NP, :] + b1it_ref[...]
    h2r = jnp.maximum(h2, 0.0)
    w = jnp.dot(h2r, kw2e_ref[...], preferred_element_type=jnp.float32)
    w_ref[...] = w
    s3_ref[...] = h2r + w + b2et_ref[...]


def _tc_v_body(s3_ref, g_ref, kw2a_ref, v_ref):
    h3 = s3_ref[...] + 0.25 * (g_ref[0, :NP, :] + g_ref[1, :NP, :])
    v_ref[...] = jnp.dot(h3, kw2a_ref[...], preferred_element_type=jnp.float32)


def _tc_out_body(v_ref, e_ref, b2it_ref, out_ref):
    h4 = v_ref[...] + e_ref[0, :NP, :] + e_ref[1, :NP, :] + b2it_ref[...]
    lane = _lane_group_iota()
    valid = lane < C
    neg = jnp.full((NP, 128), -jnp.inf, jnp.float32)
    t = jnp.where(valid, h4, neg)
    for k in (1, 2, 4, 8):
        cand = _roll_left(t, k)
        t = jnp.where(lane + k < H, jnp.maximum(t, cand), t)
    m = _group_broadcast_lane0(t, lane)
    t = h4 - m
    e = jnp.where(valid, jnp.exp(t), 0.0)
    for k in (1, 2, 4, 8):
        e = e + jnp.where(lane + k < H, _roll_left(e, k), 0.0)
    lse = jnp.log(_group_broadcast_lane0(e, lane))
    out_ref[...] = t - lse


def _tc_call(body, out_shapes, *args):
    return pl.pallas_call(
        body,
        out_shape=out_shapes,
    )(*args)


def kernel(x, edge_index, ego_edge_index, W1inter, b1inter, W1intra, b1intra,
           W2inter, b2inter, W2intra, b2intra):
    fp = jax.ShapeDtypeStruct((NP, 128), jnp.float32)

    src_e = edge_index[0].reshape(E_STREAMS, E_KK)
    dst_e = edge_index[1].reshape(E_STREAMS, E_KK)
    src_g = ego_edge_index[:, 0, :].reshape(EGO_STREAMS, EGO_KK)
    dst_g = ego_edge_index[:, 1, :].reshape(EGO_STREAMS, EGO_KK)

    eye8 = jnp.eye(8, dtype=jnp.float32)
    kw1e = jnp.kron(eye8, W1inter)
    kw1a = jnp.kron(eye8, W1intra)
    kw2e = jnp.kron(eye8, W2inter)
    kw2a = jnp.kron(eye8, jnp.pad(W2intra, ((0, 0), (0, H - C))))
    b1et = jnp.tile(b1inter, 8).reshape(1, 8 * D)
    b1it = jnp.tile(b1intra, 8).reshape(1, 128)
    b2et = jnp.tile(b2inter, 8).reshape(1, 128)
    b2it = jnp.tile(jnp.pad(b2intra, (0, H - C)), 8).reshape(1, 128)
    xp = x.reshape(NP, 8 * D)

    def sc_view(t):
        return t.reshape(N, H)

    def tc_view(g):
        return g.reshape(NC, GP, 128)

    upre, z = _tc_call(_tc_pre_body, (fp, fp), xp, kw1e, kw1a, b1et)
    gz = _get_agg(EGO_STREAMS, EGO_KK)(sc_view(z), src_g, dst_g)
    u = _tc_call(_tc_combine_body, fp, upre, tc_view(gz))

    au = _get_agg(E_STREAMS, E_KK)(sc_view(u), src_e, dst_e)
    w, s3 = _tc_call(_tc_mid_body, (fp, fp), u, tc_view(au), b1it, kw2e, b2et)

    gw = _get_agg(EGO_STREAMS, EGO_KK)(sc_view(w), src_g, dst_g)
    v = _tc_call(_tc_v_body, fp, s3, tc_view(gw), kw2a)

    av = _get_agg(E_STREAMS, E_KK)(sc_view(v), src_e, dst_e)
    outp = _tc_call(_tc_out_body, fp, v, tc_view(av), b2it)
    return outp.reshape(N, H)[:, :C]


# speedup vs baseline: 1.1531x; 1.1531x over previous
import functools

import jax
import jax.numpy as jnp
from jax import lax
from jax.experimental import pallas as pl
from jax.experimental.pallas import tpu as pltpu
from jax.experimental.pallas import tpu_sc as plsc

N = 10000
D = 128
H = 16
C = 7
E = 320000
K = 4
E_EGO = 40000

NC = 2
NS = 16
CHUNK = 128
NW = NC * NS

ACC = 10240
RPS = ACC // NS

E_STREAMS = 160
E_KK = 2000
EGO_STREAMS = 160
EGO_KK = 1000

ZUNROLL = 16


def _make_agg(nstreams, kk):
    base = nstreams // NW
    rem = nstreams - base * NW
    cap = base + (1 if rem else 0)

    @functools.partial(
        pl.kernel,
        out_type=jax.ShapeDtypeStruct((NC, ACC, H), jnp.float32),
        mesh=plsc.VectorSubcoreMesh(core_axis_name="c", subcore_axis_name="s",
                                    num_cores=NC, num_subcores=NS),
        scratch_types=[
            pltpu.VMEM((cap, kk), jnp.int32),
            pltpu.VMEM((cap, kk), jnp.int32),
            pltpu.VMEM((kk, H), jnp.float32),
            pltpu.VMEM((kk, H), jnp.float32),
            pltpu.VMEM((RPS, H), jnp.float32),
            pltpu.VMEM_SHARED((ACC, H), jnp.float32),
            pltpu.VMEM_SHARED((N, H), jnp.float32),
            pltpu.SemaphoreType.DMA,
            pltpu.SemaphoreType.DMA,
            pltpu.SemaphoreType.DMA,
            pltpu.SemaphoreType.DMA,
            pltpu.SemaphoreType.DMA,
        ],
        compiler_params=pltpu.CompilerParams(use_tc_tiling_on_sc=False),
    )
    def agg(rows_hbm, src_hbm, dst_hbm, out_hbm,
            src_v, dst_v, buf0, buf1, zbuf, acc, rows_sp, sem0, sem1, semi0, semi1, sems):
        c = lax.axis_index("c")
        s = lax.axis_index("s")
        wid = c * NS + s
        n_my = jnp.where(wid < rem, base + 1, base)

        pltpu.async_copy(src_hbm.at[pl.ds(wid * base, base)],
                         src_v.at[pl.ds(0, base)], semi0)
        pltpu.async_copy(dst_hbm.at[pl.ds(wid * base, base)],
                         dst_v.at[pl.ds(0, base)], semi1)
        nsr = N // NS
        pltpu.async_copy(rows_hbm.at[pl.ds(s * nsr, nsr)],
                         rows_sp.at[pl.ds(s * nsr, nsr)], sems)

        for i in range(ZUNROLL):
            zbuf[i] = jnp.zeros((H,), jnp.float32)

        def zero_body(i, _):
            for r in range(ZUNROLL):
                zbuf[i * ZUNROLL + r] = jnp.zeros((H,), jnp.float32)
            return 0

        lax.fori_loop(1, RPS // ZUNROLL, zero_body, 0)
        pltpu.sync_copy(zbuf, acc.at[pl.ds(s * RPS, RPS)])

        pltpu.make_async_copy(src_hbm.at[pl.ds(wid * base, base)],
                              src_v.at[pl.ds(0, base)], semi0).wait()
        pltpu.make_async_copy(dst_hbm.at[pl.ds(wid * base, base)],
                              dst_v.at[pl.ds(0, base)], semi1).wait()
        pltpu.make_async_copy(rows_hbm.at[pl.ds(s * nsr, nsr)],
                              rows_sp.at[pl.ds(s * nsr, nsr)], sems).wait()
        if rem:
            @pl.when(wid < rem)
            def _():
                pltpu.sync_copy(src_hbm.at[pl.ds(NW * base + wid, 1)],
                                src_v.at[pl.ds(base, 1)])
                pltpu.sync_copy(dst_hbm.at[pl.ds(NW * base + wid, 1)],
                                dst_v.at[pl.ds(base, 1)])

        plsc.subcore_barrier()

        def gather(j, buf, sem):
            pltpu.async_copy(rows_sp.at[src_v.at[j]], buf, sem)

        def gwait(j, buf, sem):
            pltpu.make_async_copy(rows_sp.at[src_v.at[j]], buf, sem).wait()

        def scatter(j, buf):
            pltpu.sync_copy(buf, acc.at[dst_v.at[j]], add=True)

        gather(0, buf0, sem0)

        def pair_body(p, _):
            j = 2 * p
            gather(j + 1, buf1, sem1)
            gwait(j, buf0, sem0)
            scatter(j, buf0)

            @pl.when(j + 2 < n_my)
            def _():
                gather(j + 2, buf0, sem0)

            gwait(j + 1, buf1, sem1)
            scatter(j + 1, buf1)
            return 0

        lax.fori_loop(0, n_my // 2, pair_body, 0)

        @pl.when(n_my % 2 == 1)
        def _():
            gwait(n_my - 1, buf0, sem0)
            scatter(n_my - 1, buf0)

        plsc.subcore_barrier()
        pltpu.sync_copy(acc.at[pl.ds(s * RPS, RPS)],
                        out_hbm.at[c, pl.ds(s * RPS, RPS)])

    return agg


@functools.cache
def _get_agg(nstreams, kk):
    return _make_agg(nstreams, kk)


NP = N // 8
GP = ACC // 8


def _lane_group_iota():
    return lax.broadcasted_iota(jnp.int32, (NP, 128), 1) % H


def _roll_left(x, k):
    return jnp.concatenate([x[:, k:], x[:, :k]], axis=1)


def _roll_right(x, k):
    return jnp.concatenate([x[:, -k:], x[:, :-k]], axis=1)


def _group_broadcast_lane0(s, lane):
    for k in (1, 2, 4, 8):
        s = jnp.where(lane >= k, _roll_right(s, k), s)
    return s


def _tc_pre_body(x_ref, kw1e_ref, kw1a_ref, b1et_ref, upre_ref, z_ref):
    kw1a = kw1a_ref[...]
    kp = jnp.dot(kw1e_ref[...], kw1a, preferred_element_type=jnp.float32)
    xv = x_ref[...]
    z = jnp.dot(xv, kp, preferred_element_type=jnp.float32)
    a = jnp.dot(xv, kw1a, preferred_element_type=jnp.float32)
    c1t = jnp.dot(b1et_ref[...], kw1a, preferred_element_type=jnp.float32)
    z_ref[...] = z
    upre_ref[...] = a + z + c1t


def _tc_combine_body(base_ref, g_ref, out_ref):
    out_ref[...] = base_ref[...] + 0.25 * (g_ref[0, :NP, :] + g_ref[1, :NP, :])


def _tc_mid_body(u_ref, e_ref, b1it_ref, kw2e_ref, b2et_ref, w_ref, s3_ref):
    h2 = u_ref[...] + e_ref[0, :NP, :] + e_ref[1, :NP, :] + b1it_ref[...]
    h2r = jnp.maximum(h2, 0.0)
    w = jnp.dot(h2r, kw2e_ref[...], preferred_element_type=jnp.float32)
    w_ref[...] = w
    s3_ref[...] = h2r + w + b2et_ref[...]


def _tc_v_body(s3_ref, g_ref, kw2a_ref, v_ref):
    h3 = s3_ref[...] + 0.25 * (g_ref[0, :NP, :] + g_ref[1, :NP, :])
    v_ref[...] = jnp.dot(h3, kw2a_ref[...], preferred_element_type=jnp.float32)


def _tc_out_body(v_ref, e_ref, b2it_ref, out_ref):
    h4 = v_ref[...] + e_ref[0, :NP, :] + e_ref[1, :NP, :] + b2it_ref[...]
    lane = _lane_group_iota()
    valid = lane < C
    neg = jnp.full((NP, 128), -jnp.inf, jnp.float32)
    t = jnp.where(valid, h4, neg)
    for k in (1, 2, 4, 8):
        cand = _roll_left(t, k)
        t = jnp.where(lane + k < H, jnp.maximum(t, cand), t)
    m = _group_broadcast_lane0(t, lane)
    t = h4 - m
    e = jnp.where(valid, jnp.exp(t), 0.0)
    for k in (1, 2, 4, 8):
        e = e + jnp.where(lane + k < H, _roll_left(e, k), 0.0)
    lse = jnp.log(_group_broadcast_lane0(e, lane))
    out_ref[...] = t - lse


def _tc_call(body, out_shapes, *args):
    return pl.pallas_call(
        body,
        out_shape=out_shapes,
    )(*args)


def kernel(x, edge_index, ego_edge_index, W1inter, b1inter, W1intra, b1intra,
           W2inter, b2inter, W2intra, b2intra):
    fp = jax.ShapeDtypeStruct((NP, 128), jnp.float32)

    src_e = edge_index[0].reshape(E_STREAMS, E_KK)
    dst_e = edge_index[1].reshape(E_STREAMS, E_KK)
    src_g = ego_edge_index[:, 0, :].reshape(EGO_STREAMS, EGO_KK)
    dst_g = ego_edge_index[:, 1, :].reshape(EGO_STREAMS, EGO_KK)

    eye8 = jnp.eye(8, dtype=jnp.float32)
    kw1e = jnp.kron(eye8, W1inter)
    kw1a = jnp.kron(eye8, W1intra)
    kw2e = jnp.kron(eye8, W2inter)
    kw2a = jnp.kron(eye8, jnp.pad(W2intra, ((0, 0), (0, H - C))))
    b1et = jnp.tile(b1inter, 8).reshape(1, 8 * D)
    b1it = jnp.tile(b1intra, 8).reshape(1, 128)
    b2et = jnp.tile(b2inter, 8).reshape(1, 128)
    b2it = jnp.tile(jnp.pad(b2intra, (0, H - C)), 8).reshape(1, 128)
    xp = x.reshape(NP, 8 * D)

    def sc_view(t):
        return t.reshape(N, H)

    def tc_view(g):
        return g.reshape(NC, GP, 128)

    upre, z = _tc_call(_tc_pre_body, (fp, fp), xp, kw1e, kw1a, b1et)
    gz = _get_agg(EGO_STREAMS, EGO_KK)(sc_view(z), src_g, dst_g)
    u = _tc_call(_tc_combine_body, fp, upre, tc_view(gz))

    au = _get_agg(E_STREAMS, E_KK)(sc_view(u), src_e, dst_e)
    w, s3 = _tc_call(_tc_mid_body, (fp, fp), u, tc_view(au), b1it, kw2e, b2et)

    gw = _get_agg(EGO_STREAMS, EGO_KK)(sc_view(w), src_g, dst_g)
    v = _tc_call(_tc_v_body, fp, s3, tc_view(gw), kw2a)

    av = _get_agg(E_STREAMS, E_KK)(sc_view(v), src_e, dst_e)
    outp = _tc_call(_tc_out_body, fp, v, tc_view(av), b2it)
    return outp.reshape(N, H)[:, :C]


# speedup vs baseline: 1.2574x; 1.0904x over previous
import functools

import jax
import jax.numpy as jnp
from jax import lax
from jax.experimental import pallas as pl
from jax.experimental.pallas import tpu as pltpu
from jax.experimental.pallas import tpu_sc as plsc

N = 10000
D = 128
H = 16
C = 7
E = 320000
K = 4
E_EGO = 40000

NC = 2
NS = 16
CHUNK = 128
NW = NC * NS

ACC = 10240
RPS = ACC // NS

E_STREAMS = 160
E_KK = 2000
EGO_STREAMS = 160
EGO_KK = 1000

ZUNROLL = 16


def _make_agg(nstreams, kk):
    base = nstreams // NW
    rem = nstreams - base * NW
    cap = base + (1 if rem else 0)

    @functools.partial(
        pl.kernel,
        out_type=jax.ShapeDtypeStruct((NC, ACC, H), jnp.float32),
        mesh=plsc.VectorSubcoreMesh(core_axis_name="c", subcore_axis_name="s",
                                    num_cores=NC, num_subcores=NS),
        scratch_types=[
            pltpu.VMEM((cap, kk), jnp.int32),
            pltpu.VMEM((cap, kk), jnp.int32),
            pltpu.VMEM((kk, H), jnp.float32),
            pltpu.VMEM((kk, H), jnp.float32),
            pltpu.VMEM((RPS, H), jnp.float32),
            pltpu.VMEM_SHARED((ACC, H), jnp.float32),
            pltpu.VMEM_SHARED((N, H), jnp.float32),
            pltpu.SemaphoreType.DMA,
            pltpu.SemaphoreType.DMA,
            pltpu.SemaphoreType.DMA,
            pltpu.SemaphoreType.DMA,
            pltpu.SemaphoreType.DMA,
        ],
        compiler_params=pltpu.CompilerParams(use_tc_tiling_on_sc=False),
    )
    def agg(rows_hbm, src_hbm, dst_hbm, out_hbm,
            src_v, dst_v, buf0, buf1, zbuf, acc, rows_sp, sem0, sem1, semi0, semi1, sems):
        c = lax.axis_index("c")
        s = lax.axis_index("s")
        wid = c * NS + s
        n_my = jnp.where(wid < rem, base + 1, base)

        pltpu.async_copy(src_hbm.at[pl.ds(wid * base, base)],
                         src_v.at[pl.ds(0, base)], semi0)
        pltpu.async_copy(dst_hbm.at[pl.ds(wid * base, base)],
                         dst_v.at[pl.ds(0, base)], semi1)
        nsr = N // NS
        pltpu.async_copy(rows_hbm.at[pl.ds(s * nsr, nsr)],
                         rows_sp.at[pl.ds(s * nsr, nsr)], sems)

        for i in range(ZUNROLL):
            zbuf[i] = jnp.zeros((H,), jnp.float32)

        def zero_body(i, _):
            for r in range(ZUNROLL):
                zbuf[i * ZUNROLL + r] = jnp.zeros((H,), jnp.float32)
            return 0

        lax.fori_loop(1, RPS // ZUNROLL, zero_body, 0)
        pltpu.sync_copy(zbuf, acc.at[pl.ds(s * RPS, RPS)])

        pltpu.make_async_copy(src_hbm.at[pl.ds(wid * base, base)],
                              src_v.at[pl.ds(0, base)], semi0).wait()
        pltpu.make_async_copy(dst_hbm.at[pl.ds(wid * base, base)],
                              dst_v.at[pl.ds(0, base)], semi1).wait()
        pltpu.make_async_copy(rows_hbm.at[pl.ds(s * nsr, nsr)],
                              rows_sp.at[pl.ds(s * nsr, nsr)], sems).wait()
        if rem:
            @pl.when(wid < rem)
            def _():
                pltpu.sync_copy(src_hbm.at[pl.ds(NW * base + wid, 1)],
                                src_v.at[pl.ds(base, 1)])
                pltpu.sync_copy(dst_hbm.at[pl.ds(NW * base + wid, 1)],
                                dst_v.at[pl.ds(base, 1)])

        plsc.subcore_barrier()

        def gather(j, buf, sem):
            pltpu.async_copy(rows_sp.at[src_v.at[j]], buf, sem)

        def gwait(j, buf, sem):
            pltpu.make_async_copy(rows_sp.at[src_v.at[j]], buf, sem).wait()

        def scatter(j, buf):
            pltpu.sync_copy(buf, acc.at[dst_v.at[j]], add=True)

        gather(0, buf0, sem0)

        def pair_body(p, _):
            j = 2 * p
            gather(j + 1, buf1, sem1)
            gwait(j, buf0, sem0)
            scatter(j, buf0)

            @pl.when(j + 2 < n_my)
            def _():
                gather(j + 2, buf0, sem0)

            gwait(j + 1, buf1, sem1)
            scatter(j + 1, buf1)
            return 0

        lax.fori_loop(0, n_my // 2, pair_body, 0)

        @pl.when(n_my % 2 == 1)
        def _():
            gwait(n_my - 1, buf0, sem0)
            scatter(n_my - 1, buf0)

        plsc.subcore_barrier()
        pltpu.sync_copy(acc.at[pl.ds(s * RPS, RPS)],
                        out_hbm.at[c, pl.ds(s * RPS, RPS)])

    return agg


@functools.cache
def _get_agg(nstreams, kk):
    return _make_agg(nstreams, kk)


NP = N // 8
GP = ACC // 8


def _lane_group_iota():
    return lax.broadcasted_iota(jnp.int32, (NP, 128), 1) % H


def _roll_left(x, k):
    return jnp.concatenate([x[:, k:], x[:, :k]], axis=1)


def _roll_right(x, k):
    return jnp.concatenate([x[:, -k:], x[:, :-k]], axis=1)


def _group_broadcast_lane0(s, lane):
    for k in (1, 2, 4, 8):
        s = jnp.where(lane >= k, _roll_right(s, k), s)
    return s


def _tc_pre_body(x_ref, kwp_ref, kw1a_ref, b1et_ref, upre_ref, z_ref):
    xv = x_ref[...].reshape(NP, 8 * D)
    kwp = kwp_ref[...].reshape(8 * D, 128)
    kw1a = kw1a_ref[...].reshape(8 * D, 128)
    z = jnp.dot(xv, kwp, preferred_element_type=jnp.float32)
    a = jnp.dot(xv, kw1a, preferred_element_type=jnp.float32)
    c1t = jnp.dot(b1et_ref[...], kw1a, preferred_element_type=jnp.float32)
    z_ref[...] = z
    upre_ref[...] = a + z + c1t


def _tc_combine_body(base_ref, g_ref, out_ref):
    out_ref[...] = base_ref[...] + 0.25 * (g_ref[0, :NP, :] + g_ref[1, :NP, :])


def _tc_mid_body(u_ref, e_ref, b1it_ref, kw2e_ref, b2et_ref, w_ref, s3_ref):
    h2 = u_ref[...] + e_ref[0, :NP, :] + e_ref[1, :NP, :] + b1it_ref[...]
    h2r = jnp.maximum(h2, 0.0)
    w = jnp.dot(h2r, kw2e_ref[...], preferred_element_type=jnp.float32)
    w_ref[...] = w
    s3_ref[...] = h2r + w + b2et_ref[...]


def _tc_v_body(s3_ref, g_ref, kw2a_ref, v_ref):
    h3 = s3_ref[...] + 0.25 * (g_ref[0, :NP, :] + g_ref[1, :NP, :])
    v_ref[...] = jnp.dot(h3, kw2a_ref[...], preferred_element_type=jnp.float32)


def _tc_out_body(v_ref, e_ref, b2it_ref, out_ref):
    h4 = v_ref[...] + e_ref[0, :NP, :] + e_ref[1, :NP, :] + b2it_ref[...]
    lane = _lane_group_iota()
    valid = lane < C
    neg = jnp.full((NP, 128), -jnp.inf, jnp.float32)
    t = jnp.where(valid, h4, neg)
    for k in (1, 2, 4, 8):
        cand = _roll_left(t, k)
        t = jnp.where(lane + k < H, jnp.maximum(t, cand), t)
    m = _group_broadcast_lane0(t, lane)
    t = h4 - m
    e = jnp.where(valid, jnp.exp(t), 0.0)
    for k in (1, 2, 4, 8):
        e = e + jnp.where(lane + k < H, _roll_left(e, k), 0.0)
    lse = jnp.log(_group_broadcast_lane0(e, lane))
    out_ref[...] = t - lse


def _tc_call(body, out_shapes, *args):
    return pl.pallas_call(
        body,
        out_shape=out_shapes,
    )(*args)


def kernel(x, edge_index, ego_edge_index, W1inter, b1inter, W1intra, b1intra,
           W2inter, b2inter, W2intra, b2intra):
    fp = jax.ShapeDtypeStruct((NP, 128), jnp.float32)

    src_e = edge_index[0].reshape(E_STREAMS, E_KK)
    dst_e = edge_index[1].reshape(E_STREAMS, E_KK)
    src_g = ego_edge_index[:, 0, :].reshape(EGO_STREAMS, EGO_KK)
    dst_g = ego_edge_index[:, 1, :].reshape(EGO_STREAMS, EGO_KK)

    eye8 = jnp.eye(8, dtype=jnp.float32)
    kwp = jnp.kron(eye8, W1inter @ W1intra).reshape(8, D, 128)
    kw1a = jnp.kron(eye8, W1intra).reshape(8, D, 128)
    kw2e = jnp.kron(eye8, W2inter)
    kw2a = jnp.kron(eye8, jnp.pad(W2intra, ((0, 0), (0, H - C))))
    b1et = jnp.tile(b1inter, 8).reshape(1, 8 * D)
    b1it = jnp.tile(b1intra, 8).reshape(1, 128)
    b2et = jnp.tile(b2inter, 8).reshape(1, 128)
    b2it = jnp.tile(jnp.pad(b2intra, (0, H - C)), 8).reshape(1, 128)
    xp = x.reshape(NP, 8, D)

    def sc_view(t):
        return t.reshape(N, H)

    def tc_view(g):
        return g.reshape(NC, GP, 128)

    upre, z = _tc_call(_tc_pre_body, (fp, fp), xp, kwp, kw1a, b1et)
    gz = _get_agg(EGO_STREAMS, EGO_KK)(sc_view(z), src_g, dst_g)
    u = _tc_call(_tc_combine_body, fp, upre, tc_view(gz))

    au = _get_agg(E_STREAMS, E_KK)(sc_view(u), src_e, dst_e)
    w, s3 = _tc_call(_tc_mid_body, (fp, fp), u, tc_view(au), b1it, kw2e, b2et)

    gw = _get_agg(EGO_STREAMS, EGO_KK)(sc_view(w), src_g, dst_g)
    v = _tc_call(_tc_v_body, fp, s3, tc_view(gw), kw2a)

    av = _get_agg(E_STREAMS, E_KK)(sc_view(v), src_e, dst_e)
    outp = _tc_call(_tc_out_body, fp, v, tc_view(av), b2it)
    return outp.reshape(N, H)[:, :C]
